# baseline (speedup 1.0000x reference)
import jax
import jax.numpy as jnp
from jax import lax
from jax.experimental import pallas as pl
from jax.experimental.pallas import tpu as pltpu
from jax.experimental.pallas import tpu_sc as plsc

GRID_H, GRID_W = 10, 11
NSLOT = GRID_H * GRID_W
NW = 32
NRING = 4
CHUNK = 2


def _sc_scatter(x_rows, tab_d, pz_pad, zeros_src, B, C, T, nz):
    nb = B // NW
    nm = nb // CHUNK
    rows = CHUNK * C

    mesh = plsc.VectorSubcoreMesh(core_axis_name="c", subcore_axis_name="s")

    @pl.kernel(
        out_type=jax.ShapeDtypeStruct((NSLOT * B, T), jnp.float32),
        mesh=mesh,
        scratch_types=[
            pltpu.VMEM((rows, T), jnp.float32),
            pltpu.VMEM((rows, T), jnp.float32),
            pltpu.VMEM((rows, T), jnp.float32),
            pltpu.VMEM((rows, T), jnp.float32),
            pltpu.VMEM((nm, rows), jnp.int32),
            pltpu.VMEM((48,), jnp.int32),
            pltpu.SemaphoreType.DMA,
            pltpu.SemaphoreType.DMA,
            pltpu.SemaphoreType.DMA,
            pltpu.SemaphoreType.DMA,
            pltpu.SemaphoreType.DMA,
            pltpu.SemaphoreType.DMA,
            pltpu.SemaphoreType.DMA,
            pltpu.SemaphoreType.DMA,
            pltpu.SemaphoreType.DMA,
        ],
    )
    def k(x_hbm, tab_d_hbm, pz_hbm, z_hbm, out_hbm,
          buf0, buf1, buf2, buf3, didx, pzv,
          semL0, semL1, semL2, semL3,
          semD0, semD1, semD2, semD3, semZ):
        wid = lax.axis_index("s") * 2 + lax.axis_index("c")
        b_lo = wid * nb
        x_lo = b_lo * C

        bufs = (buf0, buf1, buf2, buf3)
        semL = (semL0, semL1, semL2, semL3)
        semD = (semD0, semD1, semD2, semD3)

        pltpu.sync_copy(tab_d_hbm.at[wid], didx)
        pltpu.sync_copy(pz_hbm, pzv)

        for kz in range(nz):
            vec = pzv[pl.ds((kz // 16) * 16, 16)]
            slot = vec[kz % 16]
            dst = slot * B + b_lo
            pltpu.async_copy(z_hbm, out_hbm.at[pl.ds(dst, rows)], semZ)

        def start_load(m, j):
            return pltpu.async_copy(
                x_hbm.at[pl.ds(x_lo + m * rows, rows)], bufs[j], semL[j])

        def start_dscatter(m, j):
            return pltpu.async_copy(bufs[j], out_hbm.at[didx.at[m]], semD[j])

        def wait_load(j):
            pltpu.make_async_copy(
                x_hbm.at[pl.ds(0, rows)], bufs[j], semL[j]).wait()

        def wait_dscatter(j):
            pltpu.make_async_copy(
                bufs[j], out_hbm.at[didx.at[0]], semD[j]).wait()

        for m in range(NRING):
            start_load(m, m)
            wait_load(m)
            start_dscatter(m, m)

        @pl.loop(NRING, nm, step=NRING)
        def _(g):
            for j in range(NRING):
                m = g + j
                wait_dscatter(j)
                start_load(m, j)
                wait_load(j)
                start_dscatter(m, j)

        for j in range(NRING):
            wait_dscatter(j)

        for _kz in range(nz):
            pltpu.make_async_copy(
                z_hbm, out_hbm.at[pl.ds(0, rows)], semZ).wait()

    return k(x_rows, tab_d, pz_pad, zeros_src)


def kernel(x, channel_transformation):
    B, C, T = x.shape
    ct = channel_transformation.astype(jnp.int32)

    slots = jnp.arange(NSLOT, dtype=jnp.int32)
    occupied = (slots[:, None] == ct[None, :]).any(axis=1)
    pz = jnp.where(~occupied, size=NSLOT - C, fill_value=0)[0].astype(jnp.int32)

    rows = CHUNK * C
    nb = B // NW
    nm = nb // CHUNK
    nz = NSLOT - C
    ww = jnp.arange(NW, dtype=jnp.int32)[:, None, None]
    mm = jnp.arange(nm, dtype=jnp.int32)[None, :, None]
    ii = jnp.arange(rows, dtype=jnp.int32)[None, None, :]
    tab_d = ct[ii % C] * B + ww * nb + CHUNK * mm + ii // C
    pz_pad = jnp.concatenate([pz, pz[-2:]])

    zeros_src = jnp.zeros((rows, T), jnp.float32)
    x_rows = x.reshape(B * C, T)

    out_rows = _sc_scatter(x_rows, tab_d.astype(jnp.int32),
                           pz_pad.astype(jnp.int32), zeros_src, B, C, T, nz)
    return jnp.transpose(out_rows.reshape(GRID_H, GRID_W, B, T), (2, 0, 1, 3))


# speedup vs baseline: 17.4760x; 17.4760x over previous
import jax
import jax.numpy as jnp
from jax import lax
from jax.experimental import pallas as pl
from jax.experimental.pallas import tpu as pltpu
from jax.experimental.pallas import tpu_sc as plsc

GRID_H, GRID_W = 10, 11
NSLOT = GRID_H * GRID_W
NW = 32
NRING = 4
CHUNK = 2


def _sc_scatter(x_rows, tab_d, pz_pad, zeros_src, B, C, T, nz):
    nb = B // NW
    nm = nb // CHUNK
    rows = CHUNK * C

    mesh = plsc.VectorSubcoreMesh(core_axis_name="c", subcore_axis_name="s")

    @pl.kernel(
        out_type=jax.ShapeDtypeStruct((NSLOT * B, T), jnp.float32),
        mesh=mesh,
        scratch_types=[
            pltpu.VMEM((rows, T), jnp.float32),
            pltpu.VMEM((rows, T), jnp.float32),
            pltpu.VMEM((rows, T), jnp.float32),
            pltpu.VMEM((rows, T), jnp.float32),
            pltpu.VMEM((nm, rows), jnp.int32),
            pltpu.VMEM((48,), jnp.int32),
            pltpu.VMEM_SHARED((rows, T), jnp.float32),
            pltpu.SemaphoreType.DMA,
            pltpu.SemaphoreType.DMA,
            pltpu.SemaphoreType.DMA,
            pltpu.SemaphoreType.DMA,
            pltpu.SemaphoreType.DMA,
            pltpu.SemaphoreType.DMA,
            pltpu.SemaphoreType.DMA,
            pltpu.SemaphoreType.DMA,
            pltpu.SemaphoreType.DMA,
        ],
    )
    def k(x_hbm, tab_d_hbm, pz_hbm, z_hbm, out_hbm,
          buf0, buf1, buf2, buf3, didx, pzv, zshared,
          semL0, semL1, semL2, semL3,
          semD0, semD1, semD2, semD3, semZ):
        wid = lax.axis_index("s") * 2 + lax.axis_index("c")
        b_lo = wid * nb
        x_lo = b_lo * C

        bufs = (buf0, buf1, buf2, buf3)
        semL = (semL0, semL1, semL2, semL3)
        semD = (semD0, semD1, semD2, semD3)

        pltpu.sync_copy(tab_d_hbm.at[wid], didx)
        pltpu.sync_copy(pz_hbm, pzv)

        @pl.when(lax.axis_index("s") == 0)
        def _():
            pltpu.sync_copy(z_hbm, zshared)

        plsc.subcore_barrier()

        for kz in range(nz):
            vec = pzv[pl.ds((kz // 16) * 16, 16)]
            slot = vec[kz % 16]
            dst = slot * B + b_lo
            pltpu.async_copy(zshared, out_hbm.at[pl.ds(dst, rows)], semZ)

        def start_load(m, j):
            return pltpu.async_copy(
                x_hbm.at[pl.ds(x_lo + m * rows, rows)], bufs[j], semL[j])

        def start_dscatter(m, j):
            return pltpu.async_copy(bufs[j], out_hbm.at[didx.at[m]], semD[j])

        def wait_load(j):
            pltpu.make_async_copy(
                x_hbm.at[pl.ds(0, rows)], bufs[j], semL[j]).wait()

        def wait_dscatter(j):
            pltpu.make_async_copy(
                bufs[j], out_hbm.at[didx.at[0]], semD[j]).wait()

        for m in range(NRING):
            start_load(m, m)
            wait_load(m)
            start_dscatter(m, m)

        @pl.loop(NRING, nm, step=NRING)
        def _(g):
            for j in range(NRING):
                m = g + j
                wait_dscatter(j)
                start_load(m, j)
                wait_load(j)
                start_dscatter(m, j)

        for j in range(NRING):
            wait_dscatter(j)

        for _kz in range(nz):
            pltpu.make_async_copy(
                zshared, out_hbm.at[pl.ds(0, rows)], semZ).wait()

    return k(x_rows, tab_d, pz_pad, zeros_src)


def kernel(x, channel_transformation):
    B, C, T = x.shape
    ct = channel_transformation.astype(jnp.int32)

    slots = jnp.arange(NSLOT, dtype=jnp.int32)
    occupied = (slots[:, None] == ct[None, :]).any(axis=1)
    pz = jnp.where(~occupied, size=NSLOT - C, fill_value=0)[0].astype(jnp.int32)

    rows = CHUNK * C
    nb = B // NW
    nm = nb // CHUNK
    nz = NSLOT - C
    ww = jnp.arange(NW, dtype=jnp.int32)[:, None, None]
    mm = jnp.arange(nm, dtype=jnp.int32)[None, :, None]
    ii = jnp.arange(rows, dtype=jnp.int32)[None, None, :]
    tab_d = ct[ii % C] * B + ww * nb + CHUNK * mm + ii // C
    pz_pad = jnp.concatenate([pz, pz[-2:]])

    zeros_src = jnp.zeros((rows, T), jnp.float32)
    x_rows = x.reshape(B * C, T)

    out_rows = _sc_scatter(x_rows, tab_d.astype(jnp.int32),
                           pz_pad.astype(jnp.int32), zeros_src, B, C, T, nz)
    return jnp.transpose(out_rows.reshape(GRID_H, GRID_W, B, T), (2, 0, 1, 3))


# speedup vs baseline: 18.2400x; 1.0437x over previous
import jax
import jax.numpy as jnp
from jax import lax
from jax.experimental import pallas as pl
from jax.experimental.pallas import tpu as pltpu
from jax.experimental.pallas import tpu_sc as plsc

GRID_H, GRID_W = 10, 11
NSLOT = GRID_H * GRID_W
NW = 32
NRING = 4
CHUNK = 2


def _sc_scatter(x_rows, tab_d, pz_pad, zeros_src, B, C, T, nz):
    nb = B // NW
    nm = nb // CHUNK
    rows = CHUNK * C

    mesh = plsc.VectorSubcoreMesh(core_axis_name="c", subcore_axis_name="s")

    @pl.kernel(
        out_type=jax.ShapeDtypeStruct((NSLOT * B, T), jnp.float32),
        mesh=mesh,
        scratch_types=[
            pltpu.VMEM((rows, T), jnp.float32),
            pltpu.VMEM((rows, T), jnp.float32),
            pltpu.VMEM((rows, T), jnp.float32),
            pltpu.VMEM((rows, T), jnp.float32),
            pltpu.VMEM((nm, rows), jnp.int32),
            pltpu.VMEM((48,), jnp.int32),
            pltpu.VMEM_SHARED((rows, T), jnp.float32),
            pltpu.SemaphoreType.DMA,
            pltpu.SemaphoreType.DMA,
            pltpu.SemaphoreType.DMA,
            pltpu.SemaphoreType.DMA,
            pltpu.SemaphoreType.DMA,
            pltpu.SemaphoreType.DMA,
            pltpu.SemaphoreType.DMA,
            pltpu.SemaphoreType.DMA,
            pltpu.SemaphoreType.DMA,
        ],
    )
    def k(x_hbm, tab_d_hbm, pz_hbm, z_hbm, out_hbm,
          buf0, buf1, buf2, buf3, didx, pzv, zshared,
          semL0, semL1, semL2, semL3,
          semD0, semD1, semD2, semD3, semZ):
        wid = lax.axis_index("s") * 2 + lax.axis_index("c")
        b_lo = wid * nb
        x_lo = b_lo * C

        bufs = (buf0, buf1, buf2, buf3)
        semL = (semL0, semL1, semL2, semL3)
        semD = (semD0, semD1, semD2, semD3)

        pltpu.sync_copy(tab_d_hbm.at[wid], didx)
        pltpu.sync_copy(pz_hbm, pzv)

        @pl.when(lax.axis_index("s") == 0)
        def _():
            pltpu.sync_copy(z_hbm, zshared)

        plsc.subcore_barrier()

        for kz in range(nz):
            vec = pzv[pl.ds((kz // 16) * 16, 16)]
            slot = vec[kz % 16]
            dst = slot * B + b_lo
            pltpu.async_copy(zshared, out_hbm.at[pl.ds(dst, rows)], semZ)

        def start_load(m, j):
            return pltpu.async_copy(
                x_hbm.at[pl.ds(x_lo + m * rows, rows)], bufs[j], semL[j])

        def start_dscatter(m, j):
            return pltpu.async_copy(bufs[j], out_hbm.at[didx.at[m]], semD[j])

        def wait_load(j):
            pltpu.make_async_copy(
                x_hbm.at[pl.ds(0, rows)], bufs[j], semL[j]).wait()

        def wait_dscatter(j):
            pltpu.make_async_copy(
                bufs[j], out_hbm.at[didx.at[0]], semD[j]).wait()

        for m in range(NRING):
            start_load(m, m)
        for m in range(NRING - 2):
            wait_load(m)
            start_dscatter(m, m)

        @pl.loop(NRING, nm, step=NRING)
        def _(g):
            for j in range(NRING):
                m = g + j
                jp = (j + NRING - 2) % NRING
                wait_load(jp)
                start_dscatter(m - 2, jp)
                wait_dscatter(j)
                start_load(m, j)

        for d in (2, 1):
            jp = (NRING - d) % NRING
            wait_load(jp)
            start_dscatter(nm - d, jp)
        for j in range(NRING):
            wait_dscatter(j)

        for _kz in range(nz):
            pltpu.make_async_copy(
                zshared, out_hbm.at[pl.ds(0, rows)], semZ).wait()

    return k(x_rows, tab_d, pz_pad, zeros_src)


def kernel(x, channel_transformation):
    B, C, T = x.shape
    ct = channel_transformation.astype(jnp.int32)

    slots = jnp.arange(NSLOT, dtype=jnp.int32)
    occupied = (slots[:, None] == ct[None, :]).any(axis=1)
    pz = jnp.where(~occupied, size=NSLOT - C, fill_value=0)[0].astype(jnp.int32)

    rows = CHUNK * C
    nb = B // NW
    nm = nb // CHUNK
    nz = NSLOT - C
    ww = jnp.arange(NW, dtype=jnp.int32)[:, None, None]
    mm = jnp.arange(nm, dtype=jnp.int32)[None, :, None]
    ii = jnp.arange(rows, dtype=jnp.int32)[None, None, :]
    tab_d = ct[ii % C] * B + ww * nb + CHUNK * mm + ii // C
    pz_pad = jnp.concatenate([pz, pz[-2:]])

    zeros_src = jnp.zeros((rows, T), jnp.float32)
    x_rows = x.reshape(B * C, T)

    out_rows = _sc_scatter(x_rows, tab_d.astype(jnp.int32),
                           pz_pad.astype(jnp.int32), zeros_src, B, C, T, nz)
    return jnp.transpose(out_rows.reshape(GRID_H, GRID_W, B, T), (2, 0, 1, 3))


# speedup vs baseline: 18.3519x; 1.0061x over previous
import jax
import jax.numpy as jnp
from jax import lax
from jax.experimental import pallas as pl
from jax.experimental.pallas import tpu as pltpu
from jax.experimental.pallas import tpu_sc as plsc

GRID_H, GRID_W = 10, 11
NSLOT = GRID_H * GRID_W
NW = 32
NRING = 4
CHUNK = 2


def _sc_scatter(x_rows, tab_d, pz_pad, zeros_src, B, C, T, nz):
    nb = B // NW
    nm = nb // CHUNK
    rows = CHUNK * C

    mesh = plsc.VectorSubcoreMesh(core_axis_name="c", subcore_axis_name="s")

    @pl.kernel(
        out_type=jax.ShapeDtypeStruct((NSLOT * B, T), jnp.float32),
        mesh=mesh,
        scratch_types=[
            pltpu.VMEM((rows, T), jnp.float32),
            pltpu.VMEM((rows, T), jnp.float32),
            pltpu.VMEM((rows, T), jnp.float32),
            pltpu.VMEM((rows, T), jnp.float32),
            pltpu.VMEM((nm, rows), jnp.int32),
            pltpu.VMEM((48,), jnp.int32),
            pltpu.VMEM_SHARED((rows, T), jnp.float32),
            pltpu.SemaphoreType.DMA,
            pltpu.SemaphoreType.DMA,
            pltpu.SemaphoreType.DMA,
            pltpu.SemaphoreType.DMA,
            pltpu.SemaphoreType.DMA,
            pltpu.SemaphoreType.DMA,
            pltpu.SemaphoreType.DMA,
            pltpu.SemaphoreType.DMA,
            pltpu.SemaphoreType.DMA,
        ],
    )
    def k(x_hbm, tab_d_hbm, pz_hbm, z_hbm, out_hbm,
          buf0, buf1, buf2, buf3, didx, pzv, zshared,
          semL0, semL1, semL2, semL3,
          semD0, semD1, semD2, semD3, semZ):
        wid = lax.axis_index("s") * 2 + lax.axis_index("c")
        b_lo = wid * nb
        x_lo = b_lo * C

        bufs = (buf0, buf1, buf2, buf3)
        semL = (semL0, semL1, semL2, semL3)
        semD = (semD0, semD1, semD2, semD3)

        pltpu.sync_copy(tab_d_hbm.at[wid], didx)
        pltpu.sync_copy(pz_hbm, pzv)

        @pl.when(lax.axis_index("s") == 0)
        def _():
            pltpu.sync_copy(z_hbm, zshared)

        plsc.subcore_barrier()

        for kz in range(nz):
            vec = pzv[pl.ds((kz // 16) * 16, 16)]
            slot = vec[kz % 16]
            dst = slot * B + b_lo
            pltpu.async_copy(zshared, out_hbm.at[pl.ds(dst, rows)], semZ)

        def start_load(m, j):
            return pltpu.async_copy(
                x_hbm.at[pl.ds(x_lo + m * rows, rows)], bufs[j], semL[j])

        def start_dscatter(m, j):
            return pltpu.async_copy(bufs[j], out_hbm.at[didx.at[m]], semD[j])

        def wait_load(j):
            pltpu.make_async_copy(
                x_hbm.at[pl.ds(0, rows)], bufs[j], semL[j]).wait()

        def wait_dscatter(j):
            pltpu.make_async_copy(
                bufs[j], out_hbm.at[didx.at[0]], semD[j]).wait()

        for m in range(NRING):
            start_load(m, m)
        for m in range(NRING - 3):
            wait_load(m)
            start_dscatter(m, m)

        @pl.loop(NRING, nm, step=NRING)
        def _(g):
            for j in range(NRING):
                m = g + j
                jp = (j + NRING - 3) % NRING
                wait_load(jp)
                start_dscatter(m - 3, jp)
                wait_dscatter(j)
                start_load(m, j)

        for d in (3, 2, 1):
            jp = (NRING - d) % NRING
            wait_load(jp)
            start_dscatter(nm - d, jp)
        for j in range(NRING):
            wait_dscatter(j)

        for _kz in range(nz):
            pltpu.make_async_copy(
                zshared, out_hbm.at[pl.ds(0, rows)], semZ).wait()

    return k(x_rows, tab_d, pz_pad, zeros_src)


def kernel(x, channel_transformation):
    B, C, T = x.shape
    ct = channel_transformation.astype(jnp.int32)

    slots = jnp.arange(NSLOT, dtype=jnp.int32)
    occupied = (slots[:, None] == ct[None, :]).any(axis=1)
    pz = jnp.where(~occupied, size=NSLOT - C, fill_value=0)[0].astype(jnp.int32)

    rows = CHUNK * C
    nb = B // NW
    nm = nb // CHUNK
    nz = NSLOT - C
    ww = jnp.arange(NW, dtype=jnp.int32)[:, None, None]
    mm = jnp.arange(nm, dtype=jnp.int32)[None, :, None]
    ii = jnp.arange(rows, dtype=jnp.int32)[None, None, :]
    tab_d = ct[ii % C] * B + ww * nb + CHUNK * mm + ii // C
    pz_pad = jnp.concatenate([pz, pz[-2:]])

    zeros_src = jnp.zeros((rows, T), jnp.float32)
    x_rows = x.reshape(B * C, T)

    out_rows = _sc_scatter(x_rows, tab_d.astype(jnp.int32),
                           pz_pad.astype(jnp.int32), zeros_src, B, C, T, nz)
    return jnp.transpose(out_rows.reshape(GRID_H, GRID_W, B, T), (2, 0, 1, 3))


# speedup vs baseline: 18.4465x; 1.0052x over previous
import jax
import jax.numpy as jnp
from jax import lax
from jax.experimental import pallas as pl
from jax.experimental.pallas import tpu as pltpu
from jax.experimental.pallas import tpu_sc as plsc

GRID_H, GRID_W = 10, 11
NSLOT = GRID_H * GRID_W
NW = 32
NRING = 4
CHUNK = 2


def _sc_scatter(x3, ct_pad, pz_pad, zeros_src, B, C, T, nz):
    nb = B // NW
    nm = C
    rows = nb

    mesh = plsc.VectorSubcoreMesh(core_axis_name="c", subcore_axis_name="s")

    @pl.kernel(
        out_type=jax.ShapeDtypeStruct((NSLOT * B, T), jnp.float32),
        mesh=mesh,
        scratch_types=[
            pltpu.VMEM((rows, T), jnp.float32),
            pltpu.VMEM((rows, T), jnp.float32),
            pltpu.VMEM((rows, T), jnp.float32),
            pltpu.VMEM((rows, T), jnp.float32),
            pltpu.VMEM((64,), jnp.int32),
            pltpu.VMEM((48,), jnp.int32),
            pltpu.VMEM_SHARED((rows, T), jnp.float32),
            pltpu.SemaphoreType.DMA,
            pltpu.SemaphoreType.DMA,
            pltpu.SemaphoreType.DMA,
            pltpu.SemaphoreType.DMA,
            pltpu.SemaphoreType.DMA,
            pltpu.SemaphoreType.DMA,
            pltpu.SemaphoreType.DMA,
            pltpu.SemaphoreType.DMA,
            pltpu.SemaphoreType.DMA,
        ],
    )
    def k(x_hbm, ct_hbm, pz_hbm, z_hbm, out_hbm,
          buf0, buf1, buf2, buf3, ctv, pzv, zshared,
          semL0, semL1, semL2, semL3,
          semD0, semD1, semD2, semD3, semZ):
        wid = lax.axis_index("s") * 2 + lax.axis_index("c")
        b_lo = wid * nb
        x_lo = b_lo * C

        bufs = (buf0, buf1, buf2, buf3)
        semL = (semL0, semL1, semL2, semL3)
        semD = (semD0, semD1, semD2, semD3)

        pltpu.sync_copy(ct_hbm, ctv)
        pltpu.sync_copy(pz_hbm, pzv)

        @pl.when(lax.axis_index("s") == 0)
        def _():
            pltpu.sync_copy(z_hbm, zshared)

        plsc.subcore_barrier()

        for kz in range(nz):
            vec = pzv[pl.ds((kz // 16) * 16, 16)]
            slot = vec[kz % 16]
            dst = slot * B + b_lo
            pltpu.async_copy(zshared, out_hbm.at[pl.ds(dst, rows)], semZ)

        def slot_of(c):
            vec = ctv[pl.ds((c // 16) * 16, 16)]
            return vec[c % 16]

        def start_load(m, j):
            return pltpu.async_copy(
                x_hbm.at[pl.ds(b_lo, nb), m], bufs[j], semL[j])

        def start_dscatter(m, j):
            dst = slot_of(m) * B + b_lo
            return pltpu.async_copy(
                bufs[j], out_hbm.at[pl.ds(dst, nb)], semD[j])

        def wait_load(j):
            pltpu.make_async_copy(
                x_hbm.at[pl.ds(0, nb), 0], bufs[j], semL[j]).wait()

        def wait_dscatter(j):
            pltpu.make_async_copy(
                bufs[j], out_hbm.at[pl.ds(0, nb)], semD[j]).wait()

        for m in range(NRING):
            start_load(m, m)
        for m in range(NRING - 3):
            wait_load(m)
            start_dscatter(m, m)

        for m in range(NRING, nm):
            j = m % NRING
            jp = (m - 3) % NRING
            wait_load(jp)
            start_dscatter(m - 3, jp)
            wait_dscatter(j)
            start_load(m, j)

        for d in (3, 2, 1):
            jp = (NRING - d) % NRING
            wait_load(jp)
            start_dscatter(nm - d, jp)
        for j in range(NRING):
            wait_dscatter(j)

        for _kz in range(nz):
            pltpu.make_async_copy(
                zshared, out_hbm.at[pl.ds(0, rows)], semZ).wait()

    return k(x3, ct_pad, pz_pad, zeros_src)


def kernel(x, channel_transformation):
    B, C, T = x.shape
    ct = channel_transformation.astype(jnp.int32)

    slots = jnp.arange(NSLOT, dtype=jnp.int32)
    occupied = (slots[:, None] == ct[None, :]).any(axis=1)
    pz = jnp.where(~occupied, size=NSLOT - C, fill_value=0)[0].astype(jnp.int32)

    nb = B // NW
    nz = NSLOT - C
    pz_pad = jnp.concatenate([pz, pz[-2:]])

    zeros_src = jnp.zeros((nb, T), jnp.float32)

    out_rows = _sc_scatter(x, ct,
                           pz_pad.astype(jnp.int32), zeros_src, B, C, T, nz)
    return jnp.transpose(out_rows.reshape(GRID_H, GRID_W, B, T), (2, 0, 1, 3))
